# Initial kernel scaffold; baseline (speedup 1.0000x reference)
import jax
import jax.numpy as jnp
from jax.experimental import pallas as pl


def kernel(x, edge_index, edge_attr, W_em, b_em, W_le1, b_le1, W_nn1, b_nn1, W_ih, W_hh, b_ih, b_hh, W_nn3, b_nn3, W_dec, b_dec):
    raise NotImplementedError("write your pallas kernel here")


# speedup vs baseline: 4.3167x; 4.3167x over previous
import functools

import jax
import jax.numpy as jnp
from jax import lax
from jax.experimental import pallas as pl
from jax.experimental.pallas import tpu as pltpu
from jax.experimental.pallas import tpu_sc as plsc

H = 128
NC = 2
NS = 16
NW = NC * NS
CH = 128

_MESH = dict(core_axis_name="c", subcore_axis_name="s",
             num_cores=NC, num_subcores=NS)


def _zero_fill(buf, nwords):
    z = jnp.zeros((16,), jnp.float32)

    def body(i, _):
        buf[pl.ds(i * 16, 16)] = z
        return 0

    lax.fori_loop(0, nwords // 16, body, 0)


def _conv1_sc(x_pad, src, dst, ea, scal, *, N_pad, E, n_valid):
    SL = N_pad // NS
    PER_W = ((E + NW * CH - 1) // (NW * CH)) * CH
    MAXCH = PER_W // CH

    @functools.partial(
        pl.kernel,
        out_type=jax.ShapeDtypeStruct((NC, N_pad), jnp.float32),
        mesh=plsc.VectorSubcoreMesh(**_MESH),
        compiler_params=pltpu.CompilerParams(needs_layout_passes=False),
        scratch_types=[
            pltpu.VMEM((N_pad,), jnp.float32),
            pltpu.VMEM((CH,), jnp.int32),
            pltpu.VMEM((CH,), jnp.int32),
            pltpu.VMEM((CH,), jnp.float32),
            pltpu.VMEM((CH,), jnp.float32),
            pltpu.VMEM((SL,), jnp.float32),
            pltpu.VMEM((16,), jnp.float32),
            pltpu.VMEM_SHARED((N_pad,), jnp.float32),
        ],
    )
    def k(x_hbm, src_hbm, dst_hbm, ea_hbm, sc_hbm, out_hbm,
          xv, srcv, dstv, eav, mv, zv, scv, acc):
        c = lax.axis_index("c")
        s = lax.axis_index("s")
        w = c * NS + s
        _zero_fill(zv, SL)
        pltpu.sync_copy(zv, acc.at[pl.ds(s * SL, SL)])
        pltpu.sync_copy(x_hbm, xv)
        pltpu.sync_copy(sc_hbm, scv)
        plsc.subcore_barrier()
        scvec = scv[...]
        sK = scvec[0]
        tK = scvec[1]
        base = w * PER_W
        n_ch = jnp.minimum(MAXCH, jnp.maximum(0, (E - base) // CH))

        def chunk(i, _):
            off = base + i * CH
            pltpu.sync_copy(src_hbm.at[pl.ds(off, CH)], srcv)
            pltpu.sync_copy(dst_hbm.at[pl.ds(off, CH)], dstv)
            pltpu.sync_copy(ea_hbm.at[pl.ds(off, CH)], eav)

            def grp(g, _):
                idx = srcv[pl.ds(g * 16, 16)]
                xg = plsc.load_gather(xv, [idx])
                eag = eav[pl.ds(g * 16, 16)]
                mv[pl.ds(g * 16, 16)] = jnp.maximum(xg + eag * sK + tK, 0.0)
                return 0

            lax.fori_loop(0, CH // 16, grp, 0)
            pltpu.sync_copy(mv, acc.at[dstv], add=True)
            return 0

        lax.fori_loop(0, n_ch, chunk, 0)
        plsc.subcore_barrier()

        @pl.when(s == 0)
        def _():
            pltpu.sync_copy(acc, out_hbm.at[c])

    return k(x_pad, src, dst, ea, scal)


def _lstm_tc(x2, a0, a1, wc, *, N_pad):
    BR = 512
    grid = (N_pad // BR,)

    def body(x_ref, a0_ref, a1_ref, w_ref, h_ref, c_ref, hp_ref, hq_ref):
        a = x_ref[...] + a0_ref[...] + a1_ref[...]
        ui = w_ref[0:1, :]
        ug = w_ref[1:2, :]
        uo = w_ref[2:3, :]
        vi = w_ref[3:4, :]
        vg = w_ref[4:5, :]
        vo = w_ref[5:6, :]
        wp = w_ref[6:7, :]
        wq = w_ref[7:8, :]
        gi = jax.nn.sigmoid(a * ui + vi)
        gg = jnp.tanh(a * ug + vg)
        go = jax.nn.sigmoid(a * uo + vo)
        c_n = gi * gg
        h_n = go * jnp.tanh(c_n)
        h_ref[...] = h_n
        c_ref[...] = c_n
        hp_ref[...] = jnp.sum(h_n * wp, axis=1, keepdims=True)
        hq_ref[...] = jnp.sum(h_n * wq, axis=1, keepdims=True)

    return pl.pallas_call(
        body,
        grid=grid,
        in_specs=[
            pl.BlockSpec((BR, 1), lambda i: (i, 0)),
            pl.BlockSpec((BR, 1), lambda i: (i, 0)),
            pl.BlockSpec((BR, 1), lambda i: (i, 0)),
            pl.BlockSpec((8, H), lambda i: (0, 0)),
        ],
        out_specs=[
            pl.BlockSpec((BR, H), lambda i: (i, 0)),
            pl.BlockSpec((BR, H), lambda i: (i, 0)),
            pl.BlockSpec((BR, 1), lambda i: (i, 0)),
            pl.BlockSpec((BR, 1), lambda i: (i, 0)),
        ],
        out_shape=[
            jax.ShapeDtypeStruct((N_pad, H), jnp.float32),
            jax.ShapeDtypeStruct((N_pad, H), jnp.float32),
            jax.ShapeDtypeStruct((N_pad, 1), jnp.float32),
            jax.ShapeDtypeStruct((N_pad, 1), jnp.float32),
        ],
    )(x2, a0, a1, wc)


def _conv3_sc(h_n, src, dst, ea, wem, bem, wp, wq, *, N_pad, E):
    SL = N_pad // NS
    PER_W = ((E + NW * CH - 1) // (NW * CH)) * CH
    MAXCH = PER_W // CH

    @functools.partial(
        pl.kernel,
        out_type=[jax.ShapeDtypeStruct((NC, N_pad), jnp.float32),
                  jax.ShapeDtypeStruct((NC, N_pad), jnp.float32)],
        mesh=plsc.VectorSubcoreMesh(**_MESH),
        compiler_params=pltpu.CompilerParams(needs_layout_passes=False),
        scratch_types=[
            pltpu.VMEM((CH, H), jnp.float32),
            pltpu.VMEM((CH,), jnp.int32),
            pltpu.VMEM((CH,), jnp.int32),
            pltpu.VMEM((CH,), jnp.float32),
            pltpu.VMEM((CH,), jnp.float32),
            pltpu.VMEM((CH,), jnp.float32),
            pltpu.VMEM((H, 16), jnp.float32),
            pltpu.VMEM((H, 16), jnp.float32),
            pltpu.VMEM((H, 16), jnp.float32),
            pltpu.VMEM((H, 16), jnp.float32),
            pltpu.VMEM((SL,), jnp.float32),
            pltpu.VMEM_SHARED((N_pad,), jnp.float32),
            pltpu.VMEM_SHARED((N_pad,), jnp.float32),
            pltpu.SemaphoreType.DMA,
        ],
    )
    def k(h_hbm, src_hbm, dst_hbm, ea_hbm, wem_hbm, bem_hbm, wp_hbm, wq_hbm,
          p_out, q_out,
          rows, srcv, dstv, eav, pcv, qcv, wemv, bemv, wpv, wqv, zv,
          accp, accq, sem):
        c = lax.axis_index("c")
        s = lax.axis_index("s")
        w = c * NS + s
        _zero_fill(zv, SL)
        pltpu.sync_copy(zv, accp.at[pl.ds(s * SL, SL)])
        pltpu.sync_copy(zv, accq.at[pl.ds(s * SL, SL)])
        pltpu.sync_copy(wem_hbm, wemv)
        pltpu.sync_copy(bem_hbm, bemv)
        pltpu.sync_copy(wp_hbm, wpv)
        pltpu.sync_copy(wq_hbm, wqv)
        plsc.subcore_barrier()
        base = w * PER_W
        n_ch = jnp.minimum(MAXCH, jnp.maximum(0, (E - base) // CH))
        lanes = lax.iota(jnp.int32, 16)

        NG = CH // 16
        rowidx = tuple(lanes + g * 16 for g in range(NG))

        def chunk(i, _):
            off = base + i * CH
            pltpu.sync_copy(src_hbm.at[pl.ds(off, CH)], srcv)
            pltpu.sync_copy(dst_hbm.at[pl.ds(off, CH)], dstv)
            pltpu.sync_copy(ea_hbm.at[pl.ds(off, CH)], eav)
            pltpu.async_copy(h_hbm.at[srcv], rows, sem).wait()
            eag = tuple(eav[pl.ds(g * 16, 16)] for g in range(NG))
            z16 = jnp.zeros((16,), jnp.float32)

            def feat(kk, carry):
                pcs, qcs = carry
                wemk = wemv[kk]
                bemk = bemv[kk]
                wpk = wpv[kk]
                wqk = wqv[kk]
                colidx = jnp.full((16,), kk, jnp.int32)
                npcs = []
                nqcs = []
                for g in range(NG):
                    r = plsc.load_gather(rows, [rowidx[g], colidx])
                    t = jnp.maximum(r + eag[g] * wemk + bemk, 0.0)
                    npcs.append(pcs[g] + t * wpk)
                    nqcs.append(qcs[g] + t * wqk)
                return tuple(npcs), tuple(nqcs)

            pcs, qcs = lax.fori_loop(0, H, feat,
                                     ((z16,) * NG, (z16,) * NG), unroll=2)
            for g in range(NG):
                pcv[pl.ds(g * 16, 16)] = pcs[g]
                qcv[pl.ds(g * 16, 16)] = qcs[g]
            pltpu.sync_copy(pcv, accp.at[dstv], add=True)
            pltpu.sync_copy(qcv, accq.at[dstv], add=True)
            return 0

        lax.fori_loop(0, n_ch, chunk, 0)
        plsc.subcore_barrier()

        @pl.when(s == 0)
        def _():
            pltpu.sync_copy(accp, p_out.at[c])
            pltpu.sync_copy(accq, q_out.at[c])

    return k(h_n, src, dst, ea, wem, bem, wp, wq)


def _decode_sc(src, dst, hp, hq, p0, p1, q0, q1, scal, *, N_pad, E):
    SL = N_pad // NS
    PER_W = ((E + NW * CH - 1) // (NW * CH)) * CH
    MAXCH = PER_W // CH

    @functools.partial(
        pl.kernel,
        out_type=jax.ShapeDtypeStruct((E,), jnp.float32),
        mesh=plsc.VectorSubcoreMesh(**_MESH),
        compiler_params=pltpu.CompilerParams(needs_layout_passes=False),
        scratch_types=[
            pltpu.VMEM((N_pad,), jnp.float32),
            pltpu.VMEM((N_pad,), jnp.float32),
            pltpu.VMEM((SL,), jnp.float32),
            pltpu.VMEM((SL,), jnp.float32),
            pltpu.VMEM((SL,), jnp.float32),
            pltpu.VMEM((CH,), jnp.int32),
            pltpu.VMEM((CH,), jnp.int32),
            pltpu.VMEM((CH,), jnp.float32),
            pltpu.VMEM((16,), jnp.float32),
            pltpu.VMEM_SHARED((N_pad,), jnp.float32),
            pltpu.VMEM_SHARED((N_pad,), jnp.float32),
        ],
    )
    def k(src_hbm, dst_hbm, hp_hbm, hq_hbm, p0_hbm, p1_hbm, q0_hbm, q1_hbm,
          sc_hbm, out_hbm,
          ptab, qtab, sa, sb, sc_buf, srcv, dstv, outv, scv, psh, qsh):
        c = lax.axis_index("c")
        s = lax.axis_index("s")
        w = c * NS + s
        pltpu.sync_copy(sc_hbm, scv)
        scvec = scv[...]
        noff = s * SL

        def build(part0, part1, hx, addk, shared):
            pltpu.sync_copy(hx.at[pl.ds(noff, SL)], sa)
            pltpu.sync_copy(part0.at[pl.ds(noff, SL)], sb)
            pltpu.sync_copy(part1.at[pl.ds(noff, SL)], sc_buf)

            def body(i, _):
                j = i * 16
                sa[pl.ds(j, 16)] = (sa[pl.ds(j, 16)] + sb[pl.ds(j, 16)]
                                    + sc_buf[pl.ds(j, 16)] + addk)
                return 0

            lax.fori_loop(0, SL // 16, body, 0)
            pltpu.sync_copy(sa, shared.at[pl.ds(noff, SL)])

        build(p0_hbm, p1_hbm, hp_hbm, scvec[0], psh)
        build(q0_hbm, q1_hbm, hq_hbm, scvec[1], qsh)
        plsc.subcore_barrier()
        pltpu.sync_copy(psh, ptab)
        pltpu.sync_copy(qsh, qtab)
        base = w * PER_W
        n_ch = jnp.minimum(MAXCH, jnp.maximum(0, (E - base) // CH))

        def chunk(i, _):
            off = base + i * CH
            pltpu.sync_copy(src_hbm.at[pl.ds(off, CH)], srcv)
            pltpu.sync_copy(dst_hbm.at[pl.ds(off, CH)], dstv)

            def grp(g, _):
                pg = plsc.load_gather(ptab, [srcv[pl.ds(g * 16, 16)]])
                qg = plsc.load_gather(qtab, [dstv[pl.ds(g * 16, 16)]])
                outv[pl.ds(g * 16, 16)] = pg + qg
                return 0

            lax.fori_loop(0, CH // 16, grp, 0)
            pltpu.sync_copy(outv, out_hbm.at[pl.ds(off, CH)])
            return 0

        lax.fori_loop(0, n_ch, chunk, 0)

    return k(src, dst, hp, hq, p0, p1, q0, q1, scal)


def kernel(x, edge_index, edge_attr, W_em, b_em, W_le1, b_le1, W_nn1, b_nn1,
           W_ih, W_hh, b_ih, b_hh, W_nn3, b_nn3, W_dec, b_dec):
    N = x.shape[0]
    E = edge_index.shape[1]
    N_pad = ((N + 511) // 512) * 512
    src = edge_index[0]
    dst = edge_index[1]
    ea = edge_attr[:, 0]
    x1 = x[:, 0]
    x_pad = jnp.concatenate([x1, jnp.zeros((N_pad - N,), jnp.float32)])

    s_k = W_em[0] @ W_le1[:, 0]
    t_k = b_em @ W_le1[:, 0] + b_le1[0]
    scal_a = jnp.zeros((16,), jnp.float32).at[0].set(s_k).at[1].set(t_k)

    a_part = _conv1_sc(x_pad, src, dst, ea, scal_a, N_pad=N_pad, E=E,
                       n_valid=E)

    u = W_ih @ W_nn1[0]
    v = W_ih @ b_nn1 + b_ih + b_hh
    wp = W_nn3 @ W_dec[:H, 0]
    cp = b_nn3 @ W_dec[:H, 0]
    wq = W_nn3 @ W_dec[H:, 0]
    cq = b_nn3 @ W_dec[H:, 0]
    wc = jnp.stack([u[0:H], u[2 * H:3 * H], u[3 * H:4 * H],
                    v[0:H], v[2 * H:3 * H], v[3 * H:4 * H], wp, wq])

    h_n, c_n, hp, hq = _lstm_tc(x_pad[:, None], a_part[0][:, None],
                                a_part[1][:, None], wc, N_pad=N_pad)

    splat = lambda z: jnp.broadcast_to(z[:, None], (H, 16))
    p_part, q_part = _conv3_sc(h_n, src, dst, ea, splat(W_em[0]),
                               splat(b_em), splat(wp), splat(wq),
                               N_pad=N_pad, E=E)

    scal_e = (jnp.zeros((16,), jnp.float32)
              .at[0].set(cp).at[1].set(cq + b_dec[0]))
    out_flat = _decode_sc(src, dst, hp[:, 0], hq[:, 0],
                          p_part[0], p_part[1], q_part[0], q_part[1],
                          scal_e, N_pad=N_pad, E=E)

    return (out_flat[:, None], h_n[:N][None], c_n[:N][None])


# speedup vs baseline: 6.0631x; 1.4046x over previous
import functools

import jax
import jax.numpy as jnp
from jax import lax
from jax.experimental import pallas as pl
from jax.experimental.pallas import tpu as pltpu
from jax.experimental.pallas import tpu_sc as plsc

H = 128
NC = 2
NS = 16
NW = NC * NS
CH = 128
SK = 16

_MESH = dict(core_axis_name="c", subcore_axis_name="s",
             num_cores=NC, num_subcores=NS)
_CPARAMS = pltpu.CompilerParams(needs_layout_passes=False)


def _zero_fill(buf, nwords):
    z = jnp.zeros((16,), jnp.float32)

    def body(i, _):
        buf[pl.ds(i * 16, 16)] = z
        return 0

    lax.fori_loop(0, nwords // 16, body, 0)


def _edge_geometry(E):
    maxch = (E + NW * CH - 1) // (NW * CH)
    maxch = ((maxch + 7) // 8) * 8
    per_w = maxch * CH
    nsup = (maxch + SK - 1) // SK
    rows_pad = (NW - 1) * maxch + nsup * SK
    return per_w, maxch, nsup, rows_pad


def _suprow(i):
    return (i // SK) % 2 * SK + i % SK


def _conv1_sc(x_pad, src2, dst2, ea2, scal, *, N_pad, E):
    SL = N_pad // NS
    PER_W, MAXCH, NSUP, _ = _edge_geometry(E)

    @functools.partial(
        pl.kernel,
        out_type=jax.ShapeDtypeStruct((NC, N_pad), jnp.float32),
        mesh=plsc.VectorSubcoreMesh(**_MESH),
        compiler_params=_CPARAMS,
        scratch_types=[
            pltpu.VMEM((N_pad,), jnp.float32),
            pltpu.VMEM((2 * SK, CH), jnp.int32),
            pltpu.VMEM((2 * SK, CH), jnp.int32),
            pltpu.VMEM((2 * SK, CH), jnp.float32),
            pltpu.VMEM((2 * CH,), jnp.float32),
            pltpu.VMEM((SL,), jnp.float32),
            pltpu.VMEM((16,), jnp.float32),
            pltpu.VMEM_SHARED((N_pad,), jnp.float32),
            pltpu.SemaphoreType.DMA,
        ],
    )
    def k(x_hbm, src_hbm, dst_hbm, ea_hbm, sc_hbm, out_hbm,
          xv, srcs, dsts, eas, mv, zv, scv, acc, sem):
        c = lax.axis_index("c")
        s = lax.axis_index("s")
        w = c * NS + s
        _zero_fill(zv, SL)
        pltpu.sync_copy(zv, acc.at[pl.ds(s * SL, SL)])
        pltpu.sync_copy(x_hbm, xv)
        pltpu.sync_copy(sc_hbm, scv)
        plsc.subcore_barrier()
        scvec = scv[...]
        sK = scvec[0]
        tK = scvec[1]
        base_row = w * MAXCH
        n_ch = jnp.minimum(MAXCH, jnp.maximum(0, (E - base_row * CH) // CH))

        def loadsup(j):
            roff = (j % 2) * SK
            pltpu.sync_copy(src_hbm.at[pl.ds(base_row + j * SK, SK)],
                            srcs.at[pl.ds(roff, SK)])
            pltpu.sync_copy(dst_hbm.at[pl.ds(base_row + j * SK, SK)],
                            dsts.at[pl.ds(roff, SK)])
            pltpu.sync_copy(ea_hbm.at[pl.ds(base_row + j * SK, SK)],
                            eas.at[pl.ds(roff, SK)])

        def drain():
            pltpu.make_async_copy(mv.at[pl.ds(0, CH)],
                                  acc.at[dsts.at[0]], sem).wait()

        @pl.when(n_ch > 0)
        def _():
            loadsup(0)

        def chunk(i, _):
            nxt = i + 1

            @pl.when(jnp.logical_and(nxt < n_ch, nxt % SK == 0))
            def _():
                loadsup(nxt // SK)

            @pl.when(i >= 2)
            def _():
                drain()

            row = _suprow(i)
            boff = (i % 2) * CH

            def grp(g, _):
                idx = srcs[row, pl.ds(g * 16, 16)]
                xg = plsc.load_gather(xv, [idx])
                eag = eas[row, pl.ds(g * 16, 16)]
                mv[pl.ds(boff + g * 16, 16)] = (
                    jnp.maximum(xg + eag * sK + tK, 0.0))
                return 0

            lax.fori_loop(0, CH // 16, grp, 0)
            pltpu.async_copy(mv.at[pl.ds(boff, CH)],
                             acc.at[dsts.at[row]], sem, add=True)
            return 0

        lax.fori_loop(0, n_ch, chunk, 0)

        @pl.when(n_ch >= 1)
        def _():
            drain()

        @pl.when(n_ch >= 2)
        def _():
            drain()

        plsc.subcore_barrier()

        @pl.when(s == 0)
        def _():
            pltpu.sync_copy(acc, out_hbm.at[c])

    return k(x_pad, src2, dst2, ea2, scal)


def _lstm_tc(x2, a0, a1, wc, *, N_pad):
    BR = 512
    grid = (N_pad // BR,)

    def body(x_ref, a0_ref, a1_ref, w_ref,
             h_ref, c_ref, hb_ref, hp_ref, hq_ref):
        a = x_ref[...] + a0_ref[...] + a1_ref[...]
        ui = w_ref[0:1, :]
        ug = w_ref[1:2, :]
        uo = w_ref[2:3, :]
        vi = w_ref[3:4, :]
        vg = w_ref[4:5, :]
        vo = w_ref[5:6, :]
        wp = w_ref[6:7, :]
        wq = w_ref[7:8, :]
        bem = w_ref[8:9, :]
        gi = jax.nn.sigmoid(a * ui + vi)
        gg = jnp.tanh(a * ug + vg)
        go = jax.nn.sigmoid(a * uo + vo)
        c_n = gi * gg
        h_n = go * jnp.tanh(c_n)
        h_ref[...] = h_n
        c_ref[...] = c_n
        hb_ref[...] = h_n + bem
        hp_ref[...] = jnp.sum(h_n * wp, axis=1, keepdims=True)
        hq_ref[...] = jnp.sum(h_n * wq, axis=1, keepdims=True)

    return pl.pallas_call(
        body,
        grid=grid,
        in_specs=[
            pl.BlockSpec((BR, 1), lambda i: (i, 0)),
            pl.BlockSpec((BR, 1), lambda i: (i, 0)),
            pl.BlockSpec((BR, 1), lambda i: (i, 0)),
            pl.BlockSpec((16, H), lambda i: (0, 0)),
        ],
        out_specs=[
            pl.BlockSpec((BR, H), lambda i: (i, 0)),
            pl.BlockSpec((BR, H), lambda i: (i, 0)),
            pl.BlockSpec((BR, H), lambda i: (i, 0)),
            pl.BlockSpec((BR, 1), lambda i: (i, 0)),
            pl.BlockSpec((BR, 1), lambda i: (i, 0)),
        ],
        out_shape=[
            jax.ShapeDtypeStruct((N_pad, H), jnp.float32),
            jax.ShapeDtypeStruct((N_pad, H), jnp.float32),
            jax.ShapeDtypeStruct((N_pad, H), jnp.float32),
            jax.ShapeDtypeStruct((N_pad, 1), jnp.float32),
            jax.ShapeDtypeStruct((N_pad, 1), jnp.float32),
        ],
    )(x2, a0, a1, wc)


def _conv3_sc(hb, src2, dst2, ea2, wem, wp, wq, *, N_pad, E):
    SL = N_pad // NS
    PER_W, MAXCH, NSUP, _ = _edge_geometry(E)
    NG = CH // 16

    @functools.partial(
        pl.kernel,
        out_type=[jax.ShapeDtypeStruct((NC, N_pad), jnp.float32),
                  jax.ShapeDtypeStruct((NC, N_pad), jnp.float32)],
        mesh=plsc.VectorSubcoreMesh(**_MESH),
        compiler_params=_CPARAMS,
        scratch_types=[
            pltpu.VMEM((2 * CH, H), jnp.float32),
            pltpu.VMEM((2 * SK, CH), jnp.int32),
            pltpu.VMEM((2 * SK, CH), jnp.int32),
            pltpu.VMEM((2 * SK, CH), jnp.float32),
            pltpu.VMEM((2 * CH,), jnp.float32),
            pltpu.VMEM((2 * CH,), jnp.float32),
            pltpu.VMEM((H, 16), jnp.float32),
            pltpu.VMEM((H, 16), jnp.float32),
            pltpu.VMEM((H, 16), jnp.float32),
            pltpu.VMEM((SL,), jnp.float32),
            pltpu.VMEM_SHARED((N_pad,), jnp.float32),
            pltpu.VMEM_SHARED((N_pad,), jnp.float32),
            pltpu.SemaphoreType.DMA,
            pltpu.SemaphoreType.DMA,
        ],
    )
    def k(h_hbm, src_hbm, dst_hbm, ea_hbm, wem_hbm, wp_hbm, wq_hbm,
          p_out, q_out,
          rows, srcs, dsts, eas, pcv, qcv, wemv, wpv, wqv, zv,
          accp, accq, semg, sems):
        c = lax.axis_index("c")
        s = lax.axis_index("s")
        w = c * NS + s
        _zero_fill(zv, SL)
        pltpu.sync_copy(zv, accp.at[pl.ds(s * SL, SL)])
        pltpu.sync_copy(zv, accq.at[pl.ds(s * SL, SL)])
        pltpu.sync_copy(wem_hbm, wemv)
        pltpu.sync_copy(wp_hbm, wpv)
        pltpu.sync_copy(wq_hbm, wqv)
        plsc.subcore_barrier()
        base_row = w * MAXCH
        n_ch = jnp.minimum(MAXCH, jnp.maximum(0, (E - base_row * CH) // CH))
        lanes = lax.iota(jnp.int32, 16)
        rowidx = tuple(lanes + g * 16 for g in range(NG))

        def loadsup(j):
            roff = (j % 2) * SK
            pltpu.sync_copy(src_hbm.at[pl.ds(base_row + j * SK, SK)],
                            srcs.at[pl.ds(roff, SK)])
            pltpu.sync_copy(dst_hbm.at[pl.ds(base_row + j * SK, SK)],
                            dsts.at[pl.ds(roff, SK)])
            pltpu.sync_copy(ea_hbm.at[pl.ds(base_row + j * SK, SK)],
                            eas.at[pl.ds(roff, SK)])

        def issue_gather(i):
            pltpu.async_copy(h_hbm.at[srcs.at[_suprow(i)]],
                             rows.at[pl.ds((i % 2) * CH, CH)], semg)

        def wait_gather(i):
            pltpu.make_async_copy(h_hbm.at[srcs.at[_suprow(i)]],
                                  rows.at[pl.ds((i % 2) * CH, CH)],
                                  semg).wait()

        def drain_scatter():
            pltpu.make_async_copy(pcv.at[pl.ds(0, CH)],
                                  accp.at[dsts.at[0]], sems).wait()

        @pl.when(n_ch > 0)
        def _():
            loadsup(0)
            issue_gather(0)

        def chunk(i, _):
            nxt = i + 1

            @pl.when(jnp.logical_and(nxt < n_ch, nxt % SK == 0))
            def _():
                loadsup(nxt // SK)

            @pl.when(nxt < n_ch)
            def _():
                issue_gather(nxt)

            @pl.when(i >= 2)
            def _():
                drain_scatter()
                drain_scatter()

            wait_gather(i)
            row = _suprow(i)
            boff = (i % 2) * CH
            roff16 = jnp.full((16,), boff, jnp.int32)
            rowidx_b = tuple(rowidx[g] + roff16 for g in range(NG))
            eag = tuple(eas[row, pl.ds(g * 16, 16)] for g in range(NG))
            z16 = jnp.zeros((16,), jnp.float32)

            def feat(kk, carry):
                pcs, qcs = carry
                wemk = wemv[kk]
                wpk = wpv[kk]
                wqk = wqv[kk]
                colidx = jnp.full((16,), kk, jnp.int32)
                npcs = []
                nqcs = []
                for g in range(NG):
                    r = plsc.load_gather(rows, [rowidx_b[g], colidx])
                    t = jnp.maximum(r + eag[g] * wemk, 0.0)
                    npcs.append(pcs[g] + t * wpk)
                    nqcs.append(qcs[g] + t * wqk)
                return tuple(npcs), tuple(nqcs)

            pcs, qcs = lax.fori_loop(0, H, feat,
                                     ((z16,) * NG, (z16,) * NG), unroll=2)
            for g in range(NG):
                pcv[pl.ds(boff + g * 16, 16)] = pcs[g]
                qcv[pl.ds(boff + g * 16, 16)] = qcs[g]
            pltpu.async_copy(pcv.at[pl.ds(boff, CH)],
                             accp.at[dsts.at[row]], sems, add=True)
            pltpu.async_copy(qcv.at[pl.ds(boff, CH)],
                             accq.at[dsts.at[row]], sems, add=True)
            return 0

        lax.fori_loop(0, n_ch, chunk, 0)

        @pl.when(n_ch >= 1)
        def _():
            drain_scatter()
            drain_scatter()

        @pl.when(n_ch >= 2)
        def _():
            drain_scatter()
            drain_scatter()

        plsc.subcore_barrier()

        @pl.when(s == 0)
        def _():
            pltpu.sync_copy(accp, p_out.at[c])
            pltpu.sync_copy(accq, q_out.at[c])

    return k(hb, src2, dst2, ea2, wem, wp, wq)


def _decode_sc(src2, dst2, hp, hq, p0, p1, q0, q1, scal, *, N_pad, E):
    SL = N_pad // NS
    PER_W, MAXCH, NSUP, _ = _edge_geometry(E)

    @functools.partial(
        pl.kernel,
        out_type=jax.ShapeDtypeStruct((E // CH, CH), jnp.float32),
        mesh=plsc.VectorSubcoreMesh(**_MESH),
        compiler_params=_CPARAMS,
        scratch_types=[
            pltpu.VMEM((N_pad,), jnp.float32),
            pltpu.VMEM((N_pad,), jnp.float32),
            pltpu.VMEM((SL,), jnp.float32),
            pltpu.VMEM((SL,), jnp.float32),
            pltpu.VMEM((SL,), jnp.float32),
            pltpu.VMEM((2 * SK, CH), jnp.int32),
            pltpu.VMEM((2 * SK, CH), jnp.int32),
            pltpu.VMEM((2 * CH,), jnp.float32),
            pltpu.VMEM((16,), jnp.float32),
            pltpu.VMEM_SHARED((N_pad,), jnp.float32),
            pltpu.VMEM_SHARED((N_pad,), jnp.float32),
            pltpu.SemaphoreType.DMA,
        ],
    )
    def k(src_hbm, dst_hbm, hp_hbm, hq_hbm, p0_hbm, p1_hbm, q0_hbm, q1_hbm,
          sc_hbm, out_hbm,
          ptab, qtab, sa, sb, sc_buf, srcs, dsts, outv, scv, psh, qsh, sem):
        c = lax.axis_index("c")
        s = lax.axis_index("s")
        w = c * NS + s
        pltpu.sync_copy(sc_hbm, scv)
        scvec = scv[...]
        noff = s * SL

        def build(part0, part1, hx, addk, shared):
            pltpu.sync_copy(hx.at[pl.ds(noff, SL)], sa)
            pltpu.sync_copy(part0.at[pl.ds(noff, SL)], sb)
            pltpu.sync_copy(part1.at[pl.ds(noff, SL)], sc_buf)

            def body(i, _):
                j = i * 16
                sa[pl.ds(j, 16)] = (sa[pl.ds(j, 16)] + sb[pl.ds(j, 16)]
                                    + sc_buf[pl.ds(j, 16)] + addk)
                return 0

            lax.fori_loop(0, SL // 16, body, 0)
            pltpu.sync_copy(sa, shared.at[pl.ds(noff, SL)])

        build(p0_hbm, p1_hbm, hp_hbm, scvec[0], psh)
        build(q0_hbm, q1_hbm, hq_hbm, scvec[1], qsh)
        plsc.subcore_barrier()
        pltpu.sync_copy(psh, ptab)
        pltpu.sync_copy(qsh, qtab)
        base_row = w * MAXCH
        n_ch = jnp.minimum(MAXCH, jnp.maximum(0, (E - base_row * CH) // CH))

        def loadsup(j):
            roff = (j % 2) * SK
            pltpu.sync_copy(src_hbm.at[pl.ds(base_row + j * SK, SK)],
                            srcs.at[pl.ds(roff, SK)])
            pltpu.sync_copy(dst_hbm.at[pl.ds(base_row + j * SK, SK)],
                            dsts.at[pl.ds(roff, SK)])

        def drain_out():
            pltpu.make_async_copy(outv.at[pl.ds(0, CH)],
                                  out_hbm.at[0], sem).wait()

        @pl.when(n_ch > 0)
        def _():
            loadsup(0)

        def chunk(i, _):
            nxt = i + 1

            @pl.when(jnp.logical_and(nxt < n_ch, nxt % SK == 0))
            def _():
                loadsup(nxt // SK)

            @pl.when(i >= 2)
            def _():
                drain_out()

            row = _suprow(i)
            boff = (i % 2) * CH

            def grp(g, _):
                pg = plsc.load_gather(ptab, [srcs[row, pl.ds(g * 16, 16)]])
                qg = plsc.load_gather(qtab, [dsts[row, pl.ds(g * 16, 16)]])
                outv[pl.ds(boff + g * 16, 16)] = pg + qg
                return 0

            lax.fori_loop(0, CH // 16, grp, 0)
            pltpu.async_copy(outv.at[pl.ds(boff, CH)],
                             out_hbm.at[base_row + i], sem)
            return 0

        lax.fori_loop(0, n_ch, chunk, 0)

        @pl.when(n_ch >= 1)
        def _():
            drain_out()

        @pl.when(n_ch >= 2)
        def _():
            drain_out()

    return k(src2, dst2, hp, hq, p0, p1, q0, q1, scal)


def kernel(x, edge_index, edge_attr, W_em, b_em, W_le1, b_le1, W_nn1, b_nn1,
           W_ih, W_hh, b_ih, b_hh, W_nn3, b_nn3, W_dec, b_dec):
    N = x.shape[0]
    E = edge_index.shape[1]
    N_pad = ((N + 511) // 512) * 512
    _, _, _, rows_pad = _edge_geometry(E)
    E_pad = rows_pad * CH

    def pad2(a, dtype):
        a = jnp.concatenate([a, jnp.zeros((E_pad - E,), dtype)])
        return a.reshape(rows_pad, CH)

    src2 = pad2(edge_index[0], jnp.int32)
    dst2 = pad2(edge_index[1], jnp.int32)
    ea2 = pad2(edge_attr[:, 0], jnp.float32)
    x1 = x[:, 0]
    x_pad = jnp.concatenate([x1, jnp.zeros((N_pad - N,), jnp.float32)])

    s_k = W_em[0] @ W_le1[:, 0]
    t_k = b_em @ W_le1[:, 0] + b_le1[0]
    scal_a = jnp.zeros((16,), jnp.float32).at[0].set(s_k).at[1].set(t_k)

    a_part = _conv1_sc(x_pad, src2, dst2, ea2, scal_a, N_pad=N_pad, E=E)

    u = W_ih @ W_nn1[0]
    v = W_ih @ b_nn1 + b_ih + b_hh
    wp = W_nn3 @ W_dec[:H, 0]
    cp = b_nn3 @ W_dec[:H, 0]
    wq = W_nn3 @ W_dec[H:, 0]
    cq = b_nn3 @ W_dec[H:, 0]
    wc = jnp.zeros((16, H), jnp.float32)
    wc = wc.at[0].set(u[0:H]).at[1].set(u[2 * H:3 * H]).at[2].set(u[3 * H:])
    wc = wc.at[3].set(v[0:H]).at[4].set(v[2 * H:3 * H]).at[5].set(v[3 * H:])
    wc = wc.at[6].set(wp).at[7].set(wq).at[8].set(b_em)

    h_n, c_n, hb, hp, hq = _lstm_tc(x_pad[:, None], a_part[0][:, None],
                                    a_part[1][:, None], wc, N_pad=N_pad)

    splat = lambda z: jnp.broadcast_to(z[:, None], (H, 16))
    p_part, q_part = _conv3_sc(hb, src2, dst2, ea2, splat(W_em[0]),
                               splat(wp), splat(wq), N_pad=N_pad, E=E)

    scal_e = (jnp.zeros((16,), jnp.float32)
              .at[0].set(cp).at[1].set(cq + b_dec[0]))
    out2 = _decode_sc(src2, dst2, hp[:, 0], hq[:, 0],
                      p_part[0], p_part[1], q_part[0], q_part[1],
                      scal_e, N_pad=N_pad, E=E)

    return (out2.reshape(E, 1), h_n[:N][None], c_n[:N][None])


# speedup vs baseline: 6.0636x; 1.0001x over previous
import functools

import jax
import jax.numpy as jnp
from jax import lax
from jax.experimental import pallas as pl
from jax.experimental.pallas import tpu as pltpu
from jax.experimental.pallas import tpu_sc as plsc

H = 128
NC = 2
NS = 16
NW = NC * NS
CH = 128
SK = 16

_MESH = dict(core_axis_name="c", subcore_axis_name="s",
             num_cores=NC, num_subcores=NS)
_CPARAMS = pltpu.CompilerParams(needs_layout_passes=False)


def _zero_fill(buf, nwords):
    z = jnp.zeros((16,), jnp.float32)

    def body(i, _):
        buf[pl.ds(i * 16, 16)] = z
        return 0

    lax.fori_loop(0, nwords // 16, body, 0)


def _edge_geometry(E):
    maxch = (E + NW * CH - 1) // (NW * CH)
    maxch = ((maxch + 7) // 8) * 8
    per_w = maxch * CH
    nsup = (maxch + SK - 1) // SK
    rows_pad = (NW - 1) * maxch + nsup * SK
    return per_w, maxch, nsup, rows_pad


def _suprow(i):
    return (i // SK) % 2 * SK + i % SK


def _conv1_sc(x_pad, src2, dst2, ea2, scal, *, N_pad, E):
    SL = N_pad // NS
    PER_W, MAXCH, NSUP, _ = _edge_geometry(E)

    @functools.partial(
        pl.kernel,
        out_type=jax.ShapeDtypeStruct((NC, N_pad), jnp.float32),
        mesh=plsc.VectorSubcoreMesh(**_MESH),
        compiler_params=_CPARAMS,
        scratch_types=[
            pltpu.VMEM((N_pad,), jnp.float32),
            pltpu.VMEM((2 * SK, CH), jnp.int32),
            pltpu.VMEM((2 * SK, CH), jnp.int32),
            pltpu.VMEM((2 * SK, CH), jnp.float32),
            pltpu.VMEM((2 * CH,), jnp.float32),
            pltpu.VMEM((SL,), jnp.float32),
            pltpu.VMEM((16,), jnp.float32),
            pltpu.VMEM_SHARED((N_pad,), jnp.float32),
            pltpu.SemaphoreType.DMA,
        ],
    )
    def k(x_hbm, src_hbm, dst_hbm, ea_hbm, sc_hbm, out_hbm,
          xv, srcs, dsts, eas, mv, zv, scv, acc, sem):
        c = lax.axis_index("c")
        s = lax.axis_index("s")
        w = c * NS + s
        _zero_fill(zv, SL)
        pltpu.sync_copy(zv, acc.at[pl.ds(s * SL, SL)])
        pltpu.sync_copy(x_hbm, xv)
        pltpu.sync_copy(sc_hbm, scv)
        plsc.subcore_barrier()
        scvec = scv[...]
        sK = scvec[0]
        tK = scvec[1]
        base_row = w * MAXCH
        n_ch = jnp.minimum(MAXCH, jnp.maximum(0, (E - base_row * CH) // CH))

        def loadsup(j):
            roff = (j % 2) * SK
            pltpu.sync_copy(src_hbm.at[pl.ds(base_row + j * SK, SK)],
                            srcs.at[pl.ds(roff, SK)])
            pltpu.sync_copy(dst_hbm.at[pl.ds(base_row + j * SK, SK)],
                            dsts.at[pl.ds(roff, SK)])
            pltpu.sync_copy(ea_hbm.at[pl.ds(base_row + j * SK, SK)],
                            eas.at[pl.ds(roff, SK)])

        def drain():
            pltpu.make_async_copy(mv.at[pl.ds(0, CH)],
                                  acc.at[dsts.at[0]], sem).wait()

        @pl.when(n_ch > 0)
        def _():
            loadsup(0)

        def chunk(i, _):
            nxt = i + 1

            @pl.when(jnp.logical_and(nxt < n_ch, nxt % SK == 0))
            def _():
                loadsup(nxt // SK)

            @pl.when(i >= 2)
            def _():
                drain()

            row = _suprow(i)
            boff = (i % 2) * CH

            def grp(g, _):
                idx = srcs[row, pl.ds(g * 16, 16)]
                xg = plsc.load_gather(xv, [idx])
                eag = eas[row, pl.ds(g * 16, 16)]
                mv[pl.ds(boff + g * 16, 16)] = (
                    jnp.maximum(xg + eag * sK + tK, 0.0))
                return 0

            lax.fori_loop(0, CH // 16, grp, 0)
            pltpu.async_copy(mv.at[pl.ds(boff, CH)],
                             acc.at[dsts.at[row]], sem, add=True)
            return 0

        lax.fori_loop(0, n_ch, chunk, 0)

        @pl.when(n_ch >= 1)
        def _():
            drain()

        @pl.when(n_ch >= 2)
        def _():
            drain()

        plsc.subcore_barrier()

        @pl.when(s == 0)
        def _():
            pltpu.sync_copy(acc, out_hbm.at[c])

    return k(x_pad, src2, dst2, ea2, scal)


def _lstm_tc(x2, a0, a1, wc, *, N_pad):
    BR = 512
    grid = (N_pad // BR,)

    def body(x_ref, a0_ref, a1_ref, w_ref,
             h_ref, c_ref, hb_ref, hp_ref, hq_ref):
        a = x_ref[...] + a0_ref[...] + a1_ref[...]
        ui = w_ref[0:1, :]
        ug = w_ref[1:2, :]
        uo = w_ref[2:3, :]
        vi = w_ref[3:4, :]
        vg = w_ref[4:5, :]
        vo = w_ref[5:6, :]
        wp = w_ref[6:7, :]
        wq = w_ref[7:8, :]
        bem = w_ref[8:9, :]
        gi = jax.nn.sigmoid(a * ui + vi)
        gg = jnp.tanh(a * ug + vg)
        go = jax.nn.sigmoid(a * uo + vo)
        c_n = gi * gg
        h_n = go * jnp.tanh(c_n)
        h_ref[...] = h_n
        c_ref[...] = c_n
        hb_ref[...] = h_n + bem
        hp_ref[...] = jnp.sum(h_n * wp, axis=1, keepdims=True)
        hq_ref[...] = jnp.sum(h_n * wq, axis=1, keepdims=True)

    return pl.pallas_call(
        body,
        grid=grid,
        in_specs=[
            pl.BlockSpec((BR, 1), lambda i: (i, 0)),
            pl.BlockSpec((BR, 1), lambda i: (i, 0)),
            pl.BlockSpec((BR, 1), lambda i: (i, 0)),
            pl.BlockSpec((16, H), lambda i: (0, 0)),
        ],
        out_specs=[
            pl.BlockSpec((BR, H), lambda i: (i, 0)),
            pl.BlockSpec((BR, H), lambda i: (i, 0)),
            pl.BlockSpec((BR, H), lambda i: (i, 0)),
            pl.BlockSpec((BR, 1), lambda i: (i, 0)),
            pl.BlockSpec((BR, 1), lambda i: (i, 0)),
        ],
        out_shape=[
            jax.ShapeDtypeStruct((N_pad, H), jnp.float32),
            jax.ShapeDtypeStruct((N_pad, H), jnp.float32),
            jax.ShapeDtypeStruct((N_pad, H), jnp.float32),
            jax.ShapeDtypeStruct((N_pad, 1), jnp.float32),
            jax.ShapeDtypeStruct((N_pad, 1), jnp.float32),
        ],
    )(x2, a0, a1, wc)


def _conv3_sc(hb, src2, dst2, ea2, wem, wp, wq, *, N_pad, E):
    SL = N_pad // NS
    PER_W, MAXCH, NSUP, _ = _edge_geometry(E)
    NG = CH // 16

    @functools.partial(
        pl.kernel,
        out_type=[jax.ShapeDtypeStruct((NC, N_pad), jnp.float32),
                  jax.ShapeDtypeStruct((NC, N_pad), jnp.float32)],
        mesh=plsc.VectorSubcoreMesh(**_MESH),
        compiler_params=_CPARAMS,
        scratch_types=[
            pltpu.VMEM((3 * CH, H), jnp.float32),
            pltpu.VMEM((2 * SK, CH), jnp.int32),
            pltpu.VMEM((2 * SK, CH), jnp.int32),
            pltpu.VMEM((2 * SK, CH), jnp.float32),
            pltpu.VMEM((2 * CH,), jnp.float32),
            pltpu.VMEM((2 * CH,), jnp.float32),
            pltpu.VMEM((H, 16), jnp.float32),
            pltpu.VMEM((H, 16), jnp.float32),
            pltpu.VMEM((H, 16), jnp.float32),
            pltpu.VMEM((SL,), jnp.float32),
            pltpu.VMEM_SHARED((N_pad,), jnp.float32),
            pltpu.VMEM_SHARED((N_pad,), jnp.float32),
            pltpu.SemaphoreType.DMA,
            pltpu.SemaphoreType.DMA,
        ],
    )
    def k(h_hbm, src_hbm, dst_hbm, ea_hbm, wem_hbm, wp_hbm, wq_hbm,
          p_out, q_out,
          rows, srcs, dsts, eas, pcv, qcv, wemv, wpv, wqv, zv,
          accp, accq, semg, sems):
        c = lax.axis_index("c")
        s = lax.axis_index("s")
        w = c * NS + s
        _zero_fill(zv, SL)
        pltpu.sync_copy(zv, accp.at[pl.ds(s * SL, SL)])
        pltpu.sync_copy(zv, accq.at[pl.ds(s * SL, SL)])
        pltpu.sync_copy(wem_hbm, wemv)
        pltpu.sync_copy(wp_hbm, wpv)
        pltpu.sync_copy(wq_hbm, wqv)
        plsc.subcore_barrier()
        base_row = w * MAXCH
        n_ch = jnp.minimum(MAXCH, jnp.maximum(0, (E - base_row * CH) // CH))
        lanes = lax.iota(jnp.int32, 16)
        rowidx = tuple(lanes + g * 16 for g in range(NG))

        def loadsup(j):
            roff = (j % 2) * SK
            pltpu.sync_copy(src_hbm.at[pl.ds(base_row + j * SK, SK)],
                            srcs.at[pl.ds(roff, SK)])
            pltpu.sync_copy(dst_hbm.at[pl.ds(base_row + j * SK, SK)],
                            dsts.at[pl.ds(roff, SK)])
            pltpu.sync_copy(ea_hbm.at[pl.ds(base_row + j * SK, SK)],
                            eas.at[pl.ds(roff, SK)])

        def issue_gather(i):
            pltpu.async_copy(h_hbm.at[srcs.at[_suprow(i)]],
                             rows.at[pl.ds((i % 3) * CH, CH)], semg)

        def wait_gather(i):
            pltpu.make_async_copy(h_hbm.at[srcs.at[_suprow(i)]],
                                  rows.at[pl.ds((i % 3) * CH, CH)],
                                  semg).wait()

        def drain_scatter():
            pltpu.make_async_copy(pcv.at[pl.ds(0, CH)],
                                  accp.at[dsts.at[0]], sems).wait()

        @pl.when(n_ch > 0)
        def _():
            loadsup(0)
            for j in range(2):
                @pl.when(j < n_ch)
                def _():
                    issue_gather(j)

        def chunk(i, _):
            nxt = i + 2

            @pl.when(jnp.logical_and(nxt < n_ch, nxt % SK == 0))
            def _():
                loadsup(nxt // SK)

            @pl.when(nxt < n_ch)
            def _():
                issue_gather(nxt)

            @pl.when(i >= 2)
            def _():
                drain_scatter()
                drain_scatter()

            wait_gather(i)
            row = _suprow(i)
            boff = (i % 2) * CH
            roff16 = jnp.full((16,), boff, jnp.int32)
            rowidx_b = tuple(rowidx[g] + roff16 for g in range(NG))
            eag = tuple(eas[row, pl.ds(g * 16, 16)] for g in range(NG))
            z16 = jnp.zeros((16,), jnp.float32)

            def feat(kk, carry):
                pcs, qcs = carry
                wemk = wemv[kk]
                wpk = wpv[kk]
                wqk = wqv[kk]
                colidx = jnp.full((16,), kk, jnp.int32)
                npcs = []
                nqcs = []
                for g in range(NG):
                    r = plsc.load_gather(rows, [rowidx_b[g], colidx])
                    t = jnp.maximum(r + eag[g] * wemk, 0.0)
                    npcs.append(pcs[g] + t * wpk)
                    nqcs.append(qcs[g] + t * wqk)
                return tuple(npcs), tuple(nqcs)

            pcs, qcs = lax.fori_loop(0, H, feat,
                                     ((z16,) * NG, (z16,) * NG), unroll=2)
            for g in range(NG):
                pcv[pl.ds(boff + g * 16, 16)] = pcs[g]
                qcv[pl.ds(boff + g * 16, 16)] = qcs[g]
            pltpu.async_copy(pcv.at[pl.ds(boff, CH)],
                             accp.at[dsts.at[row]], sems, add=True)
            pltpu.async_copy(qcv.at[pl.ds(boff, CH)],
                             accq.at[dsts.at[row]], sems, add=True)
            return 0

        lax.fori_loop(0, n_ch, chunk, 0)

        @pl.when(n_ch >= 1)
        def _():
            drain_scatter()
            drain_scatter()

        @pl.when(n_ch >= 2)
        def _():
            drain_scatter()
            drain_scatter()

        plsc.subcore_barrier()

        @pl.when(s == 0)
        def _():
            pltpu.sync_copy(accp, p_out.at[c])
            pltpu.sync_copy(accq, q_out.at[c])

    return k(hb, src2, dst2, ea2, wem, wp, wq)


def _decode_sc(src2, dst2, hp, hq, p0, p1, q0, q1, scal, *, N_pad, E):
    SL = N_pad // NS
    PER_W, MAXCH, NSUP, _ = _edge_geometry(E)

    @functools.partial(
        pl.kernel,
        out_type=jax.ShapeDtypeStruct((E // CH, CH), jnp.float32),
        mesh=plsc.VectorSubcoreMesh(**_MESH),
        compiler_params=_CPARAMS,
        scratch_types=[
            pltpu.VMEM((N_pad,), jnp.float32),
            pltpu.VMEM((N_pad,), jnp.float32),
            pltpu.VMEM((SL,), jnp.float32),
            pltpu.VMEM((SL,), jnp.float32),
            pltpu.VMEM((SL,), jnp.float32),
            pltpu.VMEM((2 * SK, CH), jnp.int32),
            pltpu.VMEM((2 * SK, CH), jnp.int32),
            pltpu.VMEM((2 * CH,), jnp.float32),
            pltpu.VMEM((16,), jnp.float32),
            pltpu.VMEM_SHARED((N_pad,), jnp.float32),
            pltpu.VMEM_SHARED((N_pad,), jnp.float32),
            pltpu.SemaphoreType.DMA,
        ],
    )
    def k(src_hbm, dst_hbm, hp_hbm, hq_hbm, p0_hbm, p1_hbm, q0_hbm, q1_hbm,
          sc_hbm, out_hbm,
          ptab, qtab, sa, sb, sc_buf, srcs, dsts, outv, scv, psh, qsh, sem):
        c = lax.axis_index("c")
        s = lax.axis_index("s")
        w = c * NS + s
        pltpu.sync_copy(sc_hbm, scv)
        scvec = scv[...]
        noff = s * SL

        def build(part0, part1, hx, addk, shared):
            pltpu.sync_copy(hx.at[pl.ds(noff, SL)], sa)
            pltpu.sync_copy(part0.at[pl.ds(noff, SL)], sb)
            pltpu.sync_copy(part1.at[pl.ds(noff, SL)], sc_buf)

            def body(i, _):
                j = i * 16
                sa[pl.ds(j, 16)] = (sa[pl.ds(j, 16)] + sb[pl.ds(j, 16)]
                                    + sc_buf[pl.ds(j, 16)] + addk)
                return 0

            lax.fori_loop(0, SL // 16, body, 0)
            pltpu.sync_copy(sa, shared.at[pl.ds(noff, SL)])

        build(p0_hbm, p1_hbm, hp_hbm, scvec[0], psh)
        build(q0_hbm, q1_hbm, hq_hbm, scvec[1], qsh)
        plsc.subcore_barrier()
        pltpu.sync_copy(psh, ptab)
        pltpu.sync_copy(qsh, qtab)
        base_row = w * MAXCH
        n_ch = jnp.minimum(MAXCH, jnp.maximum(0, (E - base_row * CH) // CH))

        def loadsup(j):
            roff = (j % 2) * SK
            pltpu.sync_copy(src_hbm.at[pl.ds(base_row + j * SK, SK)],
                            srcs.at[pl.ds(roff, SK)])
            pltpu.sync_copy(dst_hbm.at[pl.ds(base_row + j * SK, SK)],
                            dsts.at[pl.ds(roff, SK)])

        def drain_out():
            pltpu.make_async_copy(outv.at[pl.ds(0, CH)],
                                  out_hbm.at[0], sem).wait()

        @pl.when(n_ch > 0)
        def _():
            loadsup(0)

        def chunk(i, _):
            nxt = i + 1

            @pl.when(jnp.logical_and(nxt < n_ch, nxt % SK == 0))
            def _():
                loadsup(nxt // SK)

            @pl.when(i >= 2)
            def _():
                drain_out()

            row = _suprow(i)
            boff = (i % 2) * CH

            def grp(g, _):
                pg = plsc.load_gather(ptab, [srcs[row, pl.ds(g * 16, 16)]])
                qg = plsc.load_gather(qtab, [dsts[row, pl.ds(g * 16, 16)]])
                outv[pl.ds(boff + g * 16, 16)] = pg + qg
                return 0

            lax.fori_loop(0, CH // 16, grp, 0)
            pltpu.async_copy(outv.at[pl.ds(boff, CH)],
                             out_hbm.at[base_row + i], sem)
            return 0

        lax.fori_loop(0, n_ch, chunk, 0)

        @pl.when(n_ch >= 1)
        def _():
            drain_out()

        @pl.when(n_ch >= 2)
        def _():
            drain_out()

    return k(src2, dst2, hp, hq, p0, p1, q0, q1, scal)


def kernel(x, edge_index, edge_attr, W_em, b_em, W_le1, b_le1, W_nn1, b_nn1,
           W_ih, W_hh, b_ih, b_hh, W_nn3, b_nn3, W_dec, b_dec):
    N = x.shape[0]
    E = edge_index.shape[1]
    N_pad = ((N + 511) // 512) * 512
    _, _, _, rows_pad = _edge_geometry(E)
    E_pad = rows_pad * CH

    def pad2(a, dtype):
        a = jnp.concatenate([a, jnp.zeros((E_pad - E,), dtype)])
        return a.reshape(rows_pad, CH)

    src2 = pad2(edge_index[0], jnp.int32)
    dst2 = pad2(edge_index[1], jnp.int32)
    ea2 = pad2(edge_attr[:, 0], jnp.float32)
    x1 = x[:, 0]
    x_pad = jnp.concatenate([x1, jnp.zeros((N_pad - N,), jnp.float32)])

    s_k = W_em[0] @ W_le1[:, 0]
    t_k = b_em @ W_le1[:, 0] + b_le1[0]
    scal_a = jnp.zeros((16,), jnp.float32).at[0].set(s_k).at[1].set(t_k)

    a_part = _conv1_sc(x_pad, src2, dst2, ea2, scal_a, N_pad=N_pad, E=E)

    u = W_ih @ W_nn1[0]
    v = W_ih @ b_nn1 + b_ih + b_hh
    wp = W_nn3 @ W_dec[:H, 0]
    cp = b_nn3 @ W_dec[:H, 0]
    wq = W_nn3 @ W_dec[H:, 0]
    cq = b_nn3 @ W_dec[H:, 0]
    wc = jnp.zeros((16, H), jnp.float32)
    wc = wc.at[0].set(u[0:H]).at[1].set(u[2 * H:3 * H]).at[2].set(u[3 * H:])
    wc = wc.at[3].set(v[0:H]).at[4].set(v[2 * H:3 * H]).at[5].set(v[3 * H:])
    wc = wc.at[6].set(wp).at[7].set(wq).at[8].set(b_em)

    h_n, c_n, hb, hp, hq = _lstm_tc(x_pad[:, None], a_part[0][:, None],
                                    a_part[1][:, None], wc, N_pad=N_pad)

    splat = lambda z: jnp.broadcast_to(z[:, None], (H, 16))
    p_part, q_part = _conv3_sc(hb, src2, dst2, ea2, splat(W_em[0]),
                               splat(wp), splat(wq), N_pad=N_pad, E=E)

    scal_e = (jnp.zeros((16,), jnp.float32)
              .at[0].set(cp).at[1].set(cq + b_dec[0]))
    out2 = _decode_sc(src2, dst2, hp[:, 0], hq[:, 0],
                      p_part[0], p_part[1], q_part[0], q_part[1],
                      scal_e, N_pad=N_pad, E=E)

    return (out2.reshape(E, 1), h_n[:N][None], c_n[:N][None])


# speedup vs baseline: 18.7987x; 3.1002x over previous
import functools

import jax
import jax.numpy as jnp
from jax import lax
from jax.experimental import pallas as pl
from jax.experimental.pallas import tpu as pltpu
from jax.experimental.pallas import tpu_sc as plsc

H = 128
NC = 2
NS = 16
NW = NC * NS
CH = 128
SK = 16

_MESH = dict(core_axis_name="c", subcore_axis_name="s",
             num_cores=NC, num_subcores=NS)
_CPARAMS = pltpu.CompilerParams(needs_layout_passes=False)


def _zero_fill(buf, nwords):
    z = jnp.zeros((16,), jnp.float32)

    def body(i, _):
        buf[pl.ds(i * 16, 16)] = z
        return 0

    lax.fori_loop(0, nwords // 16, body, 0)


def _edge_geometry(E):
    maxch = (E + NW * CH - 1) // (NW * CH)
    maxch = ((maxch + 7) // 8) * 8
    per_w = maxch * CH
    nsup = (maxch + SK - 1) // SK
    rows_pad = (NW - 1) * maxch + nsup * SK
    return per_w, maxch, nsup, rows_pad


def _suprow(i):
    return (i // SK) % 2 * SK + i % SK


def _conv1_sc(x_pad, src2, dst2, ea2, scal, *, N_pad, E):
    SL = N_pad // NS
    PER_W, MAXCH, NSUP, _ = _edge_geometry(E)

    @functools.partial(
        pl.kernel,
        out_type=jax.ShapeDtypeStruct((NC, N_pad), jnp.float32),
        mesh=plsc.VectorSubcoreMesh(**_MESH),
        compiler_params=_CPARAMS,
        scratch_types=[
            pltpu.VMEM((N_pad,), jnp.float32),
            pltpu.VMEM((2 * SK, CH), jnp.int32),
            pltpu.VMEM((2 * SK, CH), jnp.int32),
            pltpu.VMEM((2 * SK, CH), jnp.float32),
            pltpu.VMEM((2 * CH,), jnp.float32),
            pltpu.VMEM((SL,), jnp.float32),
            pltpu.VMEM((16,), jnp.float32),
            pltpu.VMEM_SHARED((N_pad,), jnp.float32),
            pltpu.SemaphoreType.DMA,
        ],
    )
    def k(x_hbm, src_hbm, dst_hbm, ea_hbm, sc_hbm, out_hbm,
          xv, srcs, dsts, eas, mv, zv, scv, acc, sem):
        c = lax.axis_index("c")
        s = lax.axis_index("s")
        w = c * NS + s
        _zero_fill(zv, SL)
        pltpu.sync_copy(zv, acc.at[pl.ds(s * SL, SL)])
        pltpu.sync_copy(x_hbm, xv)
        pltpu.sync_copy(sc_hbm, scv)
        plsc.subcore_barrier()
        scvec = scv[...]
        sK = scvec[0]
        tK = scvec[1]
        base_row = w * MAXCH
        n_ch = jnp.minimum(MAXCH, jnp.maximum(0, (E - base_row * CH) // CH))

        def loadsup(j):
            roff = (j % 2) * SK
            pltpu.sync_copy(src_hbm.at[pl.ds(base_row + j * SK, SK)],
                            srcs.at[pl.ds(roff, SK)])
            pltpu.sync_copy(dst_hbm.at[pl.ds(base_row + j * SK, SK)],
                            dsts.at[pl.ds(roff, SK)])
            pltpu.sync_copy(ea_hbm.at[pl.ds(base_row + j * SK, SK)],
                            eas.at[pl.ds(roff, SK)])

        def drain():
            pltpu.make_async_copy(mv.at[pl.ds(0, CH)],
                                  acc.at[dsts.at[0]], sem).wait()

        @pl.when(n_ch > 0)
        def _():
            loadsup(0)

        def chunk(i, _):
            nxt = i + 1

            @pl.when(jnp.logical_and(nxt < n_ch, nxt % SK == 0))
            def _():
                loadsup(nxt // SK)

            @pl.when(i >= 2)
            def _():
                drain()

            row = _suprow(i)
            boff = (i % 2) * CH

            def grp(g, _):
                idx = srcs[row, pl.ds(g * 16, 16)]
                xg = plsc.load_gather(xv, [idx])
                eag = eas[row, pl.ds(g * 16, 16)]
                mv[pl.ds(boff + g * 16, 16)] = (
                    jnp.maximum(xg + eag * sK + tK, 0.0))
                return 0

            lax.fori_loop(0, CH // 16, grp, 0)
            pltpu.async_copy(mv.at[pl.ds(boff, CH)],
                             acc.at[dsts.at[row]], sem, add=True)
            return 0

        lax.fori_loop(0, n_ch, chunk, 0)

        @pl.when(n_ch >= 1)
        def _():
            drain()

        @pl.when(n_ch >= 2)
        def _():
            drain()

        plsc.subcore_barrier()

        @pl.when(s == 0)
        def _():
            pltpu.sync_copy(acc, out_hbm.at[c])

    return k(x_pad, src2, dst2, ea2, scal)


def _lstm_tc(x2, a0, a1, wc, *, N_pad):
    BR = 512
    grid = (N_pad // BR,)

    def body(x_ref, a0_ref, a1_ref, w_ref,
             h_ref, c_ref, hb_ref, hp_ref, hq_ref):
        a = x_ref[...] + a0_ref[...] + a1_ref[...]
        ui = w_ref[0:1, :]
        ug = w_ref[1:2, :]
        uo = w_ref[2:3, :]
        vi = w_ref[3:4, :]
        vg = w_ref[4:5, :]
        vo = w_ref[5:6, :]
        wp = w_ref[6:7, :]
        wq = w_ref[7:8, :]
        bem = w_ref[8:9, :]
        gi = jax.nn.sigmoid(a * ui + vi)
        gg = jnp.tanh(a * ug + vg)
        go = jax.nn.sigmoid(a * uo + vo)
        c_n = gi * gg
        h_n = go * jnp.tanh(c_n)
        h_ref[...] = h_n
        c_ref[...] = c_n
        hb_ref[...] = h_n + bem
        hp_ref[...] = jnp.sum(h_n * wp, axis=1, keepdims=True)
        hq_ref[...] = jnp.sum(h_n * wq, axis=1, keepdims=True)

    return pl.pallas_call(
        body,
        grid=grid,
        in_specs=[
            pl.BlockSpec((BR, 1), lambda i: (i, 0)),
            pl.BlockSpec((BR, 1), lambda i: (i, 0)),
            pl.BlockSpec((BR, 1), lambda i: (i, 0)),
            pl.BlockSpec((16, H), lambda i: (0, 0)),
        ],
        out_specs=[
            pl.BlockSpec((BR, H), lambda i: (i, 0)),
            pl.BlockSpec((BR, H), lambda i: (i, 0)),
            pl.BlockSpec((BR, H), lambda i: (i, 0)),
            pl.BlockSpec((BR, 1), lambda i: (i, 0)),
            pl.BlockSpec((BR, 1), lambda i: (i, 0)),
        ],
        out_shape=[
            jax.ShapeDtypeStruct((N_pad, H), jnp.float32),
            jax.ShapeDtypeStruct((N_pad, H), jnp.float32),
            jax.ShapeDtypeStruct((N_pad, H), jnp.float32),
            jax.ShapeDtypeStruct((N_pad, 1), jnp.float32),
            jax.ShapeDtypeStruct((N_pad, 1), jnp.float32),
        ],
    )(x2, a0, a1, wc)


def _conv3_sc(hb, src2, dst2, ea2, wem, wp, wq, *, N_pad, E):
    SL = N_pad // NS
    PER_W, MAXCH, NSUP, _ = _edge_geometry(E)
    NG = CH // 16

    @functools.partial(
        pl.kernel,
        out_type=[jax.ShapeDtypeStruct((NC, N_pad), jnp.float32),
                  jax.ShapeDtypeStruct((NC, N_pad), jnp.float32)],
        mesh=plsc.VectorSubcoreMesh(**_MESH),
        compiler_params=_CPARAMS,
        scratch_types=[
            pltpu.VMEM((3 * CH, H), jnp.float32),
            pltpu.VMEM((2 * SK, CH), jnp.int32),
            pltpu.VMEM((2 * SK, CH), jnp.int32),
            pltpu.VMEM((2 * SK, CH), jnp.float32),
            pltpu.VMEM((2 * CH,), jnp.float32),
            pltpu.VMEM((2 * CH,), jnp.float32),
            pltpu.VMEM((H, 16), jnp.float32),
            pltpu.VMEM((H, 16), jnp.float32),
            pltpu.VMEM((H, 16), jnp.float32),
            pltpu.VMEM((SL,), jnp.float32),
            pltpu.VMEM_SHARED((N_pad,), jnp.float32),
            pltpu.VMEM_SHARED((N_pad,), jnp.float32),
            pltpu.SemaphoreType.DMA,
            pltpu.SemaphoreType.DMA,
        ],
    )
    def k(h_hbm, src_hbm, dst_hbm, ea_hbm, wem_hbm, wp_hbm, wq_hbm,
          p_out, q_out,
          rows, srcs, dsts, eas, pcv, qcv, wemv, wpv, wqv, zv,
          accp, accq, semg, sems):
        c = lax.axis_index("c")
        s = lax.axis_index("s")
        w = c * NS + s
        _zero_fill(zv, SL)
        pltpu.sync_copy(zv, accp.at[pl.ds(s * SL, SL)])
        pltpu.sync_copy(zv, accq.at[pl.ds(s * SL, SL)])
        pltpu.sync_copy(wem_hbm, wemv)
        pltpu.sync_copy(wp_hbm, wpv)
        pltpu.sync_copy(wq_hbm, wqv)
        plsc.subcore_barrier()
        base_row = w * MAXCH
        n_ch = jnp.minimum(MAXCH, jnp.maximum(0, (E - base_row * CH) // CH))
        lanes = lax.iota(jnp.int32, 16)
        rowidx = tuple(lanes + g * 16 for g in range(NG))

        def loadsup(j):
            roff = (j % 2) * SK
            pltpu.sync_copy(src_hbm.at[pl.ds(base_row + j * SK, SK)],
                            srcs.at[pl.ds(roff, SK)])
            pltpu.sync_copy(dst_hbm.at[pl.ds(base_row + j * SK, SK)],
                            dsts.at[pl.ds(roff, SK)])
            pltpu.sync_copy(ea_hbm.at[pl.ds(base_row + j * SK, SK)],
                            eas.at[pl.ds(roff, SK)])

        def issue_gather(i):
            pltpu.async_copy(h_hbm.at[srcs.at[_suprow(i)]],
                             rows.at[pl.ds((i % 3) * CH, CH)], semg)

        def wait_gather(i):
            pltpu.make_async_copy(h_hbm.at[srcs.at[_suprow(i)]],
                                  rows.at[pl.ds((i % 3) * CH, CH)],
                                  semg).wait()

        def drain_scatter():
            pltpu.make_async_copy(pcv.at[pl.ds(0, CH)],
                                  accp.at[dsts.at[0]], sems).wait()

        @pl.when(n_ch > 0)
        def _():
            loadsup(0)
            for j in range(2):
                @pl.when(j < n_ch)
                def _():
                    issue_gather(j)

        def chunk(i, _):
            nxt = i + 2

            @pl.when(jnp.logical_and(nxt < n_ch, nxt % SK == 0))
            def _():
                loadsup(nxt // SK)

            @pl.when(nxt < n_ch)
            def _():
                issue_gather(nxt)

            @pl.when(i >= 2)
            def _():
                drain_scatter()
                drain_scatter()

            wait_gather(i)
            row = _suprow(i)
            boff = (i % 2) * CH
            roff16 = jnp.full((16,), (i % 3) * CH, jnp.int32)
            rowidx_b = tuple(rowidx[g] + roff16 for g in range(NG))
            eag = tuple(eas[row, pl.ds(g * 16, 16)] for g in range(NG))
            z16 = jnp.zeros((16,), jnp.float32)

            def feat(kk, carry):
                pcs, qcs = carry
                wemk = wemv[kk]
                wpk = wpv[kk]
                wqk = wqv[kk]
                colidx = jnp.bitwise_and(lanes + kk, H - 1)
                npcs = []
                nqcs = []
                for g in range(NG):
                    r = plsc.load_gather(rows, [rowidx_b[g], colidx])
                    t = jnp.maximum(r + eag[g] * wemk, 0.0)
                    npcs.append(pcs[g] + t * wpk)
                    nqcs.append(qcs[g] + t * wqk)
                return tuple(npcs), tuple(nqcs)

            pcs, qcs = lax.fori_loop(0, H, feat,
                                     ((z16,) * NG, (z16,) * NG), unroll=2)
            for g in range(NG):
                pcv[pl.ds(boff + g * 16, 16)] = pcs[g]
                qcv[pl.ds(boff + g * 16, 16)] = qcs[g]
            pltpu.async_copy(pcv.at[pl.ds(boff, CH)],
                             accp.at[dsts.at[row]], sems, add=True)
            pltpu.async_copy(qcv.at[pl.ds(boff, CH)],
                             accq.at[dsts.at[row]], sems, add=True)
            return 0

        lax.fori_loop(0, n_ch, chunk, 0)

        @pl.when(n_ch >= 1)
        def _():
            drain_scatter()
            drain_scatter()

        @pl.when(n_ch >= 2)
        def _():
            drain_scatter()
            drain_scatter()

        plsc.subcore_barrier()

        @pl.when(s == 0)
        def _():
            pltpu.sync_copy(accp, p_out.at[c])
            pltpu.sync_copy(accq, q_out.at[c])

    return k(hb, src2, dst2, ea2, wem, wp, wq)


def _decode_sc(src2, dst2, hp, hq, p0, p1, q0, q1, scal, *, N_pad, E):
    SL = N_pad // NS
    PER_W, MAXCH, NSUP, _ = _edge_geometry(E)

    @functools.partial(
        pl.kernel,
        out_type=jax.ShapeDtypeStruct((E // CH, CH), jnp.float32),
        mesh=plsc.VectorSubcoreMesh(**_MESH),
        compiler_params=_CPARAMS,
        scratch_types=[
            pltpu.VMEM((N_pad,), jnp.float32),
            pltpu.VMEM((N_pad,), jnp.float32),
            pltpu.VMEM((SL,), jnp.float32),
            pltpu.VMEM((SL,), jnp.float32),
            pltpu.VMEM((SL,), jnp.float32),
            pltpu.VMEM((2 * SK, CH), jnp.int32),
            pltpu.VMEM((2 * SK, CH), jnp.int32),
            pltpu.VMEM((2 * CH,), jnp.float32),
            pltpu.VMEM((16,), jnp.float32),
            pltpu.VMEM_SHARED((N_pad,), jnp.float32),
            pltpu.VMEM_SHARED((N_pad,), jnp.float32),
            pltpu.SemaphoreType.DMA,
        ],
    )
    def k(src_hbm, dst_hbm, hp_hbm, hq_hbm, p0_hbm, p1_hbm, q0_hbm, q1_hbm,
          sc_hbm, out_hbm,
          ptab, qtab, sa, sb, sc_buf, srcs, dsts, outv, scv, psh, qsh, sem):
        c = lax.axis_index("c")
        s = lax.axis_index("s")
        w = c * NS + s
        pltpu.sync_copy(sc_hbm, scv)
        scvec = scv[...]
        noff = s * SL

        def build(part0, part1, hx, addk, shared):
            pltpu.sync_copy(hx.at[pl.ds(noff, SL)], sa)
            pltpu.sync_copy(part0.at[pl.ds(noff, SL)], sb)
            pltpu.sync_copy(part1.at[pl.ds(noff, SL)], sc_buf)

            def body(i, _):
                j = i * 16
                sa[pl.ds(j, 16)] = (sa[pl.ds(j, 16)] + sb[pl.ds(j, 16)]
                                    + sc_buf[pl.ds(j, 16)] + addk)
                return 0

            lax.fori_loop(0, SL // 16, body, 0)
            pltpu.sync_copy(sa, shared.at[pl.ds(noff, SL)])

        build(p0_hbm, p1_hbm, hp_hbm, scvec[0], psh)
        build(q0_hbm, q1_hbm, hq_hbm, scvec[1], qsh)
        plsc.subcore_barrier()
        pltpu.sync_copy(psh, ptab)
        pltpu.sync_copy(qsh, qtab)
        base_row = w * MAXCH
        n_ch = jnp.minimum(MAXCH, jnp.maximum(0, (E - base_row * CH) // CH))

        def loadsup(j):
            roff = (j % 2) * SK
            pltpu.sync_copy(src_hbm.at[pl.ds(base_row + j * SK, SK)],
                            srcs.at[pl.ds(roff, SK)])
            pltpu.sync_copy(dst_hbm.at[pl.ds(base_row + j * SK, SK)],
                            dsts.at[pl.ds(roff, SK)])

        def drain_out():
            pltpu.make_async_copy(outv.at[pl.ds(0, CH)],
                                  out_hbm.at[0], sem).wait()

        @pl.when(n_ch > 0)
        def _():
            loadsup(0)

        def chunk(i, _):
            nxt = i + 1

            @pl.when(jnp.logical_and(nxt < n_ch, nxt % SK == 0))
            def _():
                loadsup(nxt // SK)

            @pl.when(i >= 2)
            def _():
                drain_out()

            row = _suprow(i)
            boff = (i % 2) * CH

            def grp(g, _):
                pg = plsc.load_gather(ptab, [srcs[row, pl.ds(g * 16, 16)]])
                qg = plsc.load_gather(qtab, [dsts[row, pl.ds(g * 16, 16)]])
                outv[pl.ds(boff + g * 16, 16)] = pg + qg
                return 0

            lax.fori_loop(0, CH // 16, grp, 0)
            pltpu.async_copy(outv.at[pl.ds(boff, CH)],
                             out_hbm.at[base_row + i], sem)
            return 0

        lax.fori_loop(0, n_ch, chunk, 0)

        @pl.when(n_ch >= 1)
        def _():
            drain_out()

        @pl.when(n_ch >= 2)
        def _():
            drain_out()

    return k(src2, dst2, hp, hq, p0, p1, q0, q1, scal)


def kernel(x, edge_index, edge_attr, W_em, b_em, W_le1, b_le1, W_nn1, b_nn1,
           W_ih, W_hh, b_ih, b_hh, W_nn3, b_nn3, W_dec, b_dec):
    N = x.shape[0]
    E = edge_index.shape[1]
    N_pad = ((N + 511) // 512) * 512
    _, _, _, rows_pad = _edge_geometry(E)
    E_pad = rows_pad * CH

    def pad2(a, dtype):
        a = jnp.concatenate([a, jnp.zeros((E_pad - E,), dtype)])
        return a.reshape(rows_pad, CH)

    src2 = pad2(edge_index[0], jnp.int32)
    dst2 = pad2(edge_index[1], jnp.int32)
    ea2 = pad2(edge_attr[:, 0], jnp.float32)
    x1 = x[:, 0]
    x_pad = jnp.concatenate([x1, jnp.zeros((N_pad - N,), jnp.float32)])

    s_k = W_em[0] @ W_le1[:, 0]
    t_k = b_em @ W_le1[:, 0] + b_le1[0]
    scal_a = jnp.zeros((16,), jnp.float32).at[0].set(s_k).at[1].set(t_k)

    a_part = _conv1_sc(x_pad, src2, dst2, ea2, scal_a, N_pad=N_pad, E=E)

    u = W_ih @ W_nn1[0]
    v = W_ih @ b_nn1 + b_ih + b_hh
    wp = W_nn3 @ W_dec[:H, 0]
    cp = b_nn3 @ W_dec[:H, 0]
    wq = W_nn3 @ W_dec[H:, 0]
    cq = b_nn3 @ W_dec[H:, 0]
    wc = jnp.zeros((16, H), jnp.float32)
    wc = wc.at[0].set(u[0:H]).at[1].set(u[2 * H:3 * H]).at[2].set(u[3 * H:])
    wc = wc.at[3].set(v[0:H]).at[4].set(v[2 * H:3 * H]).at[5].set(v[3 * H:])
    wc = wc.at[6].set(wp).at[7].set(wq).at[8].set(b_em)

    h_n, c_n, hb, hp, hq = _lstm_tc(x_pad[:, None], a_part[0][:, None],
                                    a_part[1][:, None], wc, N_pad=N_pad)

    rot = (jnp.arange(H)[:, None] + jnp.arange(16)[None, :]) % H
    p_part, q_part = _conv3_sc(hb, src2, dst2, ea2, W_em[0][rot],
                               wp[rot], wq[rot], N_pad=N_pad, E=E)

    scal_e = (jnp.zeros((16,), jnp.float32)
              .at[0].set(cp).at[1].set(cq + b_dec[0]))
    out2 = _decode_sc(src2, dst2, hp[:, 0], hq[:, 0],
                      p_part[0], p_part[1], q_part[0], q_part[1],
                      scal_e, N_pad=N_pad, E=E)

    return (out2.reshape(E, 1), h_n[:N][None], c_n[:N][None])


# speedup vs baseline: 18.9550x; 1.0083x over previous
import functools

import jax
import jax.numpy as jnp
from jax import lax
from jax.experimental import pallas as pl
from jax.experimental.pallas import tpu as pltpu
from jax.experimental.pallas import tpu_sc as plsc

H = 128
NC = 2
NS = 16
NW = NC * NS
CH = 128
SK = 16

_MESH = dict(core_axis_name="c", subcore_axis_name="s",
             num_cores=NC, num_subcores=NS)
_CPARAMS = pltpu.CompilerParams(needs_layout_passes=False)


def _zero_fill(buf, nwords):
    z = jnp.zeros((16,), jnp.float32)

    def body(i, _):
        buf[pl.ds(i * 16, 16)] = z
        return 0

    lax.fori_loop(0, nwords // 16, body, 0)


def _edge_geometry(E):
    maxch = (E + NW * CH - 1) // (NW * CH)
    maxch = ((maxch + 7) // 8) * 8
    per_w = maxch * CH
    nsup = (maxch + SK - 1) // SK
    rows_pad = (NW - 1) * maxch + nsup * SK
    return per_w, maxch, nsup, rows_pad


def _suprow(i):
    return (i // SK) % 2 * SK + i % SK


def _conv1_sc(x_pad, src2, dst2, ea2, scal, *, N_pad, E):
    SL = N_pad // NS
    PER_W, MAXCH, NSUP, _ = _edge_geometry(E)

    @functools.partial(
        pl.kernel,
        out_type=jax.ShapeDtypeStruct((NC, N_pad), jnp.float32),
        mesh=plsc.VectorSubcoreMesh(**_MESH),
        compiler_params=_CPARAMS,
        scratch_types=[
            pltpu.VMEM((N_pad,), jnp.float32),
            pltpu.VMEM((2 * SK, CH), jnp.int32),
            pltpu.VMEM((2 * SK, CH), jnp.int32),
            pltpu.VMEM((2 * SK, CH), jnp.float32),
            pltpu.VMEM((2 * CH,), jnp.float32),
            pltpu.VMEM((SL,), jnp.float32),
            pltpu.VMEM((16,), jnp.float32),
            pltpu.VMEM_SHARED((N_pad,), jnp.float32),
            pltpu.SemaphoreType.DMA,
        ],
    )
    def k(x_hbm, src_hbm, dst_hbm, ea_hbm, sc_hbm, out_hbm,
          xv, srcs, dsts, eas, mv, zv, scv, acc, sem):
        c = lax.axis_index("c")
        s = lax.axis_index("s")
        w = c * NS + s
        _zero_fill(zv, SL)
        pltpu.sync_copy(zv, acc.at[pl.ds(s * SL, SL)])
        pltpu.sync_copy(x_hbm, xv)
        pltpu.sync_copy(sc_hbm, scv)
        plsc.subcore_barrier()
        scvec = scv[...]
        sK = scvec[0]
        tK = scvec[1]
        base_row = w * MAXCH
        n_ch = jnp.minimum(MAXCH, jnp.maximum(0, (E - base_row * CH) // CH))

        def loadsup(j):
            roff = (j % 2) * SK
            pltpu.sync_copy(src_hbm.at[pl.ds(base_row + j * SK, SK)],
                            srcs.at[pl.ds(roff, SK)])
            pltpu.sync_copy(dst_hbm.at[pl.ds(base_row + j * SK, SK)],
                            dsts.at[pl.ds(roff, SK)])
            pltpu.sync_copy(ea_hbm.at[pl.ds(base_row + j * SK, SK)],
                            eas.at[pl.ds(roff, SK)])

        def drain():
            pltpu.make_async_copy(mv.at[pl.ds(0, CH)],
                                  acc.at[dsts.at[0]], sem).wait()

        @pl.when(n_ch > 0)
        def _():
            loadsup(0)

        def chunk(i, _):
            nxt = i + 1

            @pl.when(jnp.logical_and(nxt < n_ch, nxt % SK == 0))
            def _():
                loadsup(nxt // SK)

            @pl.when(i >= 2)
            def _():
                drain()

            row = _suprow(i)
            boff = (i % 2) * CH

            def grp(g, _):
                idx = srcs[row, pl.ds(g * 16, 16)]
                xg = plsc.load_gather(xv, [idx])
                eag = eas[row, pl.ds(g * 16, 16)]
                mv[pl.ds(boff + g * 16, 16)] = (
                    jnp.maximum(xg + eag * sK + tK, 0.0))
                return 0

            lax.fori_loop(0, CH // 16, grp, 0)
            pltpu.async_copy(mv.at[pl.ds(boff, CH)],
                             acc.at[dsts.at[row]], sem, add=True)
            return 0

        lax.fori_loop(0, n_ch, chunk, 0)

        @pl.when(n_ch >= 1)
        def _():
            drain()

        @pl.when(n_ch >= 2)
        def _():
            drain()

        plsc.subcore_barrier()

        @pl.when(s == 0)
        def _():
            pltpu.sync_copy(acc, out_hbm.at[c])

    return k(x_pad, src2, dst2, ea2, scal)


def _lstm_tc(x2, a0, a1, wc, *, N_pad, N):
    BR = 512
    grid = (N_pad // BR,)

    def body(x_ref, a0_ref, a1_ref, w_ref,
             h_ref, c_ref, hb_ref, hp_ref, hq_ref):
        a = x_ref[...] + a0_ref[...] + a1_ref[...]
        ui = w_ref[0:1, :]
        ug = w_ref[1:2, :]
        uo = w_ref[2:3, :]
        vi = w_ref[3:4, :]
        vg = w_ref[4:5, :]
        vo = w_ref[5:6, :]
        wp = w_ref[6:7, :]
        wq = w_ref[7:8, :]
        bem = w_ref[8:9, :]
        gi = jax.nn.sigmoid(a * ui + vi)
        gg = jnp.tanh(a * ug + vg)
        go = jax.nn.sigmoid(a * uo + vo)
        c_n = gi * gg
        h_n = go * jnp.tanh(c_n)
        h_ref[...] = h_n
        c_ref[...] = c_n
        hb_ref[...] = h_n + bem
        hp_ref[...] = jnp.sum(h_n * wp, axis=1, keepdims=True)
        hq_ref[...] = jnp.sum(h_n * wq, axis=1, keepdims=True)

    return pl.pallas_call(
        body,
        grid=grid,
        in_specs=[
            pl.BlockSpec((BR, 1), lambda i: (i, 0)),
            pl.BlockSpec((BR, 1), lambda i: (i, 0)),
            pl.BlockSpec((BR, 1), lambda i: (i, 0)),
            pl.BlockSpec((16, H), lambda i: (0, 0)),
        ],
        out_specs=[
            pl.BlockSpec((BR, H), lambda i: (i, 0)),
            pl.BlockSpec((BR, H), lambda i: (i, 0)),
            pl.BlockSpec((BR, H), lambda i: (i, 0)),
            pl.BlockSpec((BR, 1), lambda i: (i, 0)),
            pl.BlockSpec((BR, 1), lambda i: (i, 0)),
        ],
        out_shape=[
            jax.ShapeDtypeStruct((N, H), jnp.float32),
            jax.ShapeDtypeStruct((N, H), jnp.float32),
            jax.ShapeDtypeStruct((N, H), jnp.float32),
            jax.ShapeDtypeStruct((N_pad, 1), jnp.float32),
            jax.ShapeDtypeStruct((N_pad, 1), jnp.float32),
        ],
    )(x2, a0, a1, wc)


def _conv3_sc(hb, src2, dst2, ea2, wem, wp, wq, *, N_pad, E):
    SL = N_pad // NS
    PER_W, MAXCH, NSUP, _ = _edge_geometry(E)
    NG = CH // 16

    @functools.partial(
        pl.kernel,
        out_type=[jax.ShapeDtypeStruct((NC, N_pad), jnp.float32),
                  jax.ShapeDtypeStruct((NC, N_pad), jnp.float32)],
        mesh=plsc.VectorSubcoreMesh(**_MESH),
        compiler_params=_CPARAMS,
        scratch_types=[
            pltpu.VMEM((3 * CH, H), jnp.float32),
            pltpu.VMEM((2 * SK, CH), jnp.int32),
            pltpu.VMEM((2 * SK, CH), jnp.int32),
            pltpu.VMEM((2 * SK, CH), jnp.float32),
            pltpu.VMEM((2 * CH,), jnp.float32),
            pltpu.VMEM((2 * CH,), jnp.float32),
            pltpu.VMEM((H, 16), jnp.float32),
            pltpu.VMEM((H, 16), jnp.float32),
            pltpu.VMEM((H, 16), jnp.float32),
            pltpu.VMEM((SL,), jnp.float32),
            pltpu.VMEM_SHARED((N_pad,), jnp.float32),
            pltpu.VMEM_SHARED((N_pad,), jnp.float32),
            pltpu.SemaphoreType.DMA,
            pltpu.SemaphoreType.DMA,
        ],
    )
    def k(h_hbm, src_hbm, dst_hbm, ea_hbm, wem_hbm, wp_hbm, wq_hbm,
          p_out, q_out,
          rows, srcs, dsts, eas, pcv, qcv, wemv, wpv, wqv, zv,
          accp, accq, semg, sems):
        c = lax.axis_index("c")
        s = lax.axis_index("s")
        w = c * NS + s
        _zero_fill(zv, SL)
        pltpu.sync_copy(zv, accp.at[pl.ds(s * SL, SL)])
        pltpu.sync_copy(zv, accq.at[pl.ds(s * SL, SL)])
        pltpu.sync_copy(wem_hbm, wemv)
        pltpu.sync_copy(wp_hbm, wpv)
        pltpu.sync_copy(wq_hbm, wqv)
        plsc.subcore_barrier()
        base_row = w * MAXCH
        n_ch = jnp.minimum(MAXCH, jnp.maximum(0, (E - base_row * CH) // CH))
        lanes = lax.iota(jnp.int32, 16)
        rowidx = tuple(lanes + g * 16 for g in range(NG))

        def loadsup(j):
            roff = (j % 2) * SK
            pltpu.sync_copy(src_hbm.at[pl.ds(base_row + j * SK, SK)],
                            srcs.at[pl.ds(roff, SK)])
            pltpu.sync_copy(dst_hbm.at[pl.ds(base_row + j * SK, SK)],
                            dsts.at[pl.ds(roff, SK)])
            pltpu.sync_copy(ea_hbm.at[pl.ds(base_row + j * SK, SK)],
                            eas.at[pl.ds(roff, SK)])

        def issue_gather(i):
            pltpu.async_copy(h_hbm.at[srcs.at[_suprow(i)]],
                             rows.at[pl.ds((i % 3) * CH, CH)], semg)

        def wait_gather(i):
            pltpu.make_async_copy(h_hbm.at[srcs.at[_suprow(i)]],
                                  rows.at[pl.ds((i % 3) * CH, CH)],
                                  semg).wait()

        def drain_scatter():
            pltpu.make_async_copy(pcv.at[pl.ds(0, CH)],
                                  accp.at[dsts.at[0]], sems).wait()

        @pl.when(n_ch > 0)
        def _():
            loadsup(0)
            for j in range(2):
                @pl.when(j < n_ch)
                def _():
                    issue_gather(j)

        def chunk(i, _):
            nxt = i + 2

            @pl.when(jnp.logical_and(nxt < n_ch, nxt % SK == 0))
            def _():
                loadsup(nxt // SK)

            @pl.when(nxt < n_ch)
            def _():
                issue_gather(nxt)

            @pl.when(i >= 2)
            def _():
                drain_scatter()
                drain_scatter()

            wait_gather(i)
            row = _suprow(i)
            boff = (i % 2) * CH
            roff16 = jnp.full((16,), (i % 3) * CH, jnp.int32)
            rowidx_b = tuple(rowidx[g] + roff16 for g in range(NG))
            eag = tuple(eas[row, pl.ds(g * 16, 16)] for g in range(NG))
            z16 = jnp.zeros((16,), jnp.float32)

            def feat(kk, carry):
                pcs, qcs = carry
                wemk = wemv[kk]
                wpk = wpv[kk]
                wqk = wqv[kk]
                colidx = jnp.bitwise_and(lanes + kk, H - 1)
                npcs = []
                nqcs = []
                for g in range(NG):
                    r = plsc.load_gather(rows, [rowidx_b[g], colidx])
                    t = jnp.maximum(r + eag[g] * wemk, 0.0)
                    npcs.append(pcs[g] + t * wpk)
                    nqcs.append(qcs[g] + t * wqk)
                return tuple(npcs), tuple(nqcs)

            pcs, qcs = lax.fori_loop(0, H, feat,
                                     ((z16,) * NG, (z16,) * NG), unroll=4)
            for g in range(NG):
                pcv[pl.ds(boff + g * 16, 16)] = pcs[g]
                qcv[pl.ds(boff + g * 16, 16)] = qcs[g]
            pltpu.async_copy(pcv.at[pl.ds(boff, CH)],
                             accp.at[dsts.at[row]], sems, add=True)
            pltpu.async_copy(qcv.at[pl.ds(boff, CH)],
                             accq.at[dsts.at[row]], sems, add=True)
            return 0

        lax.fori_loop(0, n_ch, chunk, 0)

        @pl.when(n_ch >= 1)
        def _():
            drain_scatter()
            drain_scatter()

        @pl.when(n_ch >= 2)
        def _():
            drain_scatter()
            drain_scatter()

        plsc.subcore_barrier()

        @pl.when(s == 0)
        def _():
            pltpu.sync_copy(accp, p_out.at[c])
            pltpu.sync_copy(accq, q_out.at[c])

    return k(hb, src2, dst2, ea2, wem, wp, wq)


def _decode_sc(src2, dst2, hp, hq, p0, p1, q0, q1, scal, *, N_pad, E):
    SL = N_pad // NS
    PER_W, MAXCH, NSUP, _ = _edge_geometry(E)

    @functools.partial(
        pl.kernel,
        out_type=jax.ShapeDtypeStruct((E // CH, CH), jnp.float32),
        mesh=plsc.VectorSubcoreMesh(**_MESH),
        compiler_params=_CPARAMS,
        scratch_types=[
            pltpu.VMEM((N_pad,), jnp.float32),
            pltpu.VMEM((N_pad,), jnp.float32),
            pltpu.VMEM((SL,), jnp.float32),
            pltpu.VMEM((SL,), jnp.float32),
            pltpu.VMEM((SL,), jnp.float32),
            pltpu.VMEM((2 * SK, CH), jnp.int32),
            pltpu.VMEM((2 * SK, CH), jnp.int32),
            pltpu.VMEM((2 * CH,), jnp.float32),
            pltpu.VMEM((16,), jnp.float32),
            pltpu.VMEM_SHARED((N_pad,), jnp.float32),
            pltpu.VMEM_SHARED((N_pad,), jnp.float32),
            pltpu.SemaphoreType.DMA,
        ],
    )
    def k(src_hbm, dst_hbm, hp_hbm, hq_hbm, p0_hbm, p1_hbm, q0_hbm, q1_hbm,
          sc_hbm, out_hbm,
          ptab, qtab, sa, sb, sc_buf, srcs, dsts, outv, scv, psh, qsh, sem):
        c = lax.axis_index("c")
        s = lax.axis_index("s")
        w = c * NS + s
        pltpu.sync_copy(sc_hbm, scv)
        scvec = scv[...]
        noff = s * SL

        def build(part0, part1, hx, addk, shared):
            pltpu.sync_copy(hx.at[pl.ds(noff, SL)], sa)
            pltpu.sync_copy(part0.at[pl.ds(noff, SL)], sb)
            pltpu.sync_copy(part1.at[pl.ds(noff, SL)], sc_buf)

            def body(i, _):
                j = i * 16
                sa[pl.ds(j, 16)] = (sa[pl.ds(j, 16)] + sb[pl.ds(j, 16)]
                                    + sc_buf[pl.ds(j, 16)] + addk)
                return 0

            lax.fori_loop(0, SL // 16, body, 0)
            pltpu.sync_copy(sa, shared.at[pl.ds(noff, SL)])

        build(p0_hbm, p1_hbm, hp_hbm, scvec[0], psh)
        build(q0_hbm, q1_hbm, hq_hbm, scvec[1], qsh)
        plsc.subcore_barrier()
        pltpu.sync_copy(psh, ptab)
        pltpu.sync_copy(qsh, qtab)
        base_row = w * MAXCH
        n_ch = jnp.minimum(MAXCH, jnp.maximum(0, (E - base_row * CH) // CH))

        def loadsup(j):
            roff = (j % 2) * SK
            pltpu.sync_copy(src_hbm.at[pl.ds(base_row + j * SK, SK)],
                            srcs.at[pl.ds(roff, SK)])
            pltpu.sync_copy(dst_hbm.at[pl.ds(base_row + j * SK, SK)],
                            dsts.at[pl.ds(roff, SK)])

        def drain_out():
            pltpu.make_async_copy(outv.at[pl.ds(0, CH)],
                                  out_hbm.at[0], sem).wait()

        @pl.when(n_ch > 0)
        def _():
            loadsup(0)

        def chunk(i, _):
            nxt = i + 1

            @pl.when(jnp.logical_and(nxt < n_ch, nxt % SK == 0))
            def _():
                loadsup(nxt // SK)

            @pl.when(i >= 2)
            def _():
                drain_out()

            row = _suprow(i)
            boff = (i % 2) * CH

            def grp(g, _):
                pg = plsc.load_gather(ptab, [srcs[row, pl.ds(g * 16, 16)]])
                qg = plsc.load_gather(qtab, [dsts[row, pl.ds(g * 16, 16)]])
                outv[pl.ds(boff + g * 16, 16)] = pg + qg
                return 0

            lax.fori_loop(0, CH // 16, grp, 0)
            pltpu.async_copy(outv.at[pl.ds(boff, CH)],
                             out_hbm.at[base_row + i], sem)
            return 0

        lax.fori_loop(0, n_ch, chunk, 0)

        @pl.when(n_ch >= 1)
        def _():
            drain_out()

        @pl.when(n_ch >= 2)
        def _():
            drain_out()

    return k(src2, dst2, hp, hq, p0, p1, q0, q1, scal)


def kernel(x, edge_index, edge_attr, W_em, b_em, W_le1, b_le1, W_nn1, b_nn1,
           W_ih, W_hh, b_ih, b_hh, W_nn3, b_nn3, W_dec, b_dec):
    N = x.shape[0]
    E = edge_index.shape[1]
    N_pad = ((N + 511) // 512) * 512
    _, _, _, rows_pad = _edge_geometry(E)
    E_pad = rows_pad * CH

    def pad2(a, dtype):
        a = jnp.concatenate([a, jnp.zeros((E_pad - E,), dtype)])
        return a.reshape(rows_pad, CH)

    src2 = pad2(edge_index[0], jnp.int32)
    dst2 = pad2(edge_index[1], jnp.int32)
    ea2 = pad2(edge_attr[:, 0], jnp.float32)
    x1 = x[:, 0]
    x_pad = jnp.concatenate([x1, jnp.zeros((N_pad - N,), jnp.float32)])

    s_k = W_em[0] @ W_le1[:, 0]
    t_k = b_em @ W_le1[:, 0] + b_le1[0]
    scal_a = jnp.zeros((16,), jnp.float32).at[0].set(s_k).at[1].set(t_k)

    a_part = _conv1_sc(x_pad, src2, dst2, ea2, scal_a, N_pad=N_pad, E=E)

    u = W_ih @ W_nn1[0]
    v = W_ih @ b_nn1 + b_ih + b_hh
    wp = W_nn3 @ W_dec[:H, 0]
    cp = b_nn3 @ W_dec[:H, 0]
    wq = W_nn3 @ W_dec[H:, 0]
    cq = b_nn3 @ W_dec[H:, 0]
    wc = jnp.zeros((16, H), jnp.float32)
    wc = wc.at[0].set(u[0:H]).at[1].set(u[2 * H:3 * H]).at[2].set(u[3 * H:])
    wc = wc.at[3].set(v[0:H]).at[4].set(v[2 * H:3 * H]).at[5].set(v[3 * H:])
    wc = wc.at[6].set(wp).at[7].set(wq).at[8].set(b_em)

    h_n, c_n, hb, hp, hq = _lstm_tc(x_pad[:, None], a_part[0][:, None],
                                    a_part[1][:, None], wc, N_pad=N_pad, N=N)

    rot = (jnp.arange(H)[:, None] + jnp.arange(16)[None, :]) % H
    p_part, q_part = _conv3_sc(hb, src2, dst2, ea2, W_em[0][rot],
                               wp[rot], wq[rot], N_pad=N_pad, E=E)

    scal_e = (jnp.zeros((16,), jnp.float32)
              .at[0].set(cp).at[1].set(cq + b_dec[0]))
    out2 = _decode_sc(src2, dst2, hp[:, 0], hq[:, 0],
                      p_part[0], p_part[1], q_part[0], q_part[1],
                      scal_e, N_pad=N_pad, E=E)

    return (out2.reshape(E, 1), h_n[None], c_n[None])
